# baseline (speedup 1.0000x reference)
import functools

import jax
import jax.numpy as jnp
import numpy as np
from jax import lax
from jax.experimental import pallas as pl
from jax.experimental.pallas import tpu as pltpu
from jax.experimental.pallas import tpu_sc as plsc

N_PTS_ = 128
MIN_DEPTH_ = 2.0
MAX_DEPTH_ = 6.0
STEP_ = (MAX_DEPTH_ - MIN_DEPTH_) / N_PTS_
BM = 4096

_NC = 2
_NS = 16
_NW = _NC * _NS
_TR = 256


def _tc_body(odt_ref, s_ref, pts_ref):
    odt = odt_ref[...]
    s = s_ref[...]
    flat = jax.lax.dot_general(
        odt, s, (((0,), (0,)), ((), ())),
        preferred_element_type=jnp.float32,
        precision=jax.lax.Precision.DEFAULT)
    for c in range(3):
        pts_ref[c, :, :] = flat[:, c * N_PTS_:(c + 1) * N_PTS_]


def _sc_lens_body(out_hbm, tile_v, sem):
    wid = lax.axis_index("s") * _NC + lax.axis_index("c")
    rows_per_w = out_hbm.shape[0] // _NW
    base = wid * rows_per_w

    lane = lax.iota(jnp.int32, 16).astype(jnp.float32)
    zvs = [MIN_DEPTH_ + (lane + 16.0 * j) * STEP_ for j in range(8)]

    def fill_row(r, carry):
        for j in range(8):
            tile_v[r, pl.ds(16 * j, 16)] = zvs[j]
        return carry

    lax.fori_loop(0, _TR, fill_row, 0)

    n_dma = rows_per_w // _TR
    copies = [
        pltpu.async_copy(
            tile_v, out_hbm.at[pl.ds(base + t * _TR, _TR)], sem)
        for t in range(n_dma)
    ]
    for c in copies:
        c.wait()


@functools.partial(jax.jit, static_argnums=())
def kernel(origins, directions):
    m = origins.shape[0]
    z = np.arange(MIN_DEPTH_, MAX_DEPTH_, STEP_, dtype=np.float32)
    s = np.zeros((6, 3 * N_PTS_), dtype=np.float32)
    for c in range(3):
        s[c, c * N_PTS_:(c + 1) * N_PTS_] = 1.0
        s[3 + c, c * N_PTS_:(c + 1) * N_PTS_] = z
    s = jnp.asarray(s)

    odt = jnp.concatenate([origins.T, directions.T], axis=0)

    grid = (m // BM,)
    pts_t = pl.pallas_call(
        _tc_body,
        grid=grid,
        in_specs=[
            pl.BlockSpec((6, BM), lambda i: (0, i)),
            pl.BlockSpec((6, 3 * N_PTS_), lambda i: (0, 0)),
        ],
        out_specs=pl.BlockSpec((3, BM, N_PTS_), lambda i: (0, i, 0)),
        out_shape=jax.ShapeDtypeStruct((3, m, N_PTS_), jnp.float32),
        compiler_params=pltpu.CompilerParams(
            dimension_semantics=("parallel",)),
    )(odt, s)

    lens_fn = functools.partial(
        pl.kernel,
        mesh=plsc.VectorSubcoreMesh(core_axis_name="c", subcore_axis_name="s"),
        out_type=jax.ShapeDtypeStruct((m, N_PTS_), jnp.float32),
        scratch_types=[
            pltpu.VMEM((_TR, N_PTS_), jnp.float32),
            pltpu.SemaphoreType.DMA,
        ],
    )(_sc_lens_body)
    lens = lens_fn()

    return (jnp.transpose(pts_t, (1, 2, 0)), lens.reshape(m, N_PTS_, 1))
